# baseline (speedup 1.0000x reference)
import functools

import numpy as np
import jax
import jax.numpy as jnp
from jax import lax
from jax.experimental import pallas as pl
from jax.experimental.pallas import tpu as pltpu
from jax.experimental.pallas import tpu_sc as plsc

NSF = 16
EDGE_FEAT = 48
HID = 48
WNUM = 384
N_NODES = 50000
N_EDGES = 800000
EB = 4000
NB = 2000
CHA = 1000
CHC = 400
ROWS_PER_TILE = N_NODES // 16

_PERM = np.empty(WNUM, np.int32)
for _u in range(16):
    for _t in range(24):
        if _t < 16:
            _src = _u * 16 + _t
        elif _t < 20:
            _src = 256 + _u * 4 + (_t - 16)
        else:
            _src = 320 + _u * 4 + (_t - 20)
        _PERM[_u * 24 + _t] = _src


def _gather_body(node_hbm, dst_hbm, x_hbm, idx_v, rows_v, sem):
    c = lax.axis_index("c")
    s = lax.axis_index("s")
    wid = s * 2 + c
    per_w = N_EDGES // 32
    base = wid * per_w

    def step(i, _):
        off = base + i * CHA
        pltpu.sync_copy(dst_hbm.at[pl.ds(off, CHA)], idx_v)
        pltpu.async_copy(node_hbm.at[idx_v], rows_v, sem).wait()
        pltpu.sync_copy(rows_v, x_hbm.at[pl.ds(off, CHA)])
        return ()

    lax.fori_loop(0, per_w // CHA, step, ())


def _sc_gather(node_attr, edge_dst):
    mesh = plsc.VectorSubcoreMesh(core_axis_name="c", subcore_axis_name="s")
    k = functools.partial(
        pl.kernel,
        out_type=jax.ShapeDtypeStruct((N_EDGES, NSF), jnp.float32),
        mesh=mesh,
        scratch_types=[
            pltpu.VMEM((CHA,), jnp.int32),
            pltpu.VMEM((CHA, NSF), jnp.float32),
            pltpu.SemaphoreType.DMA,
        ],
        compiler_params=pltpu.CompilerParams(use_tc_tiling_on_sc=False),
    )(_gather_body)
    return k(node_attr, edge_dst)


_R = np.zeros((16, 384), np.float32)
_S = np.zeros((384, 24), np.float32)
for _u in range(16):
    for _t in range(24):
        _R[_u, _u * 24 + _t] = 1.0
        _S[_u * 24 + _t, _t] = 0.25
_P1 = np.zeros((2, 24, 32), np.float32)
_P2 = np.zeros((2, 9, 32), np.float32)
for _m in range(16):
    _P1[0, _m, _m] = 1.0
    _P2[0, 0, _m] = 1.0
for _m in range(12):
    _half, _col = (0, 16 + _m) if _m < 8 else (1, _m - 8)
    _P1[_half, 16 + _m // 3, _col] = 1.0
    _P2[_half, 1 + _m % 3, _col] = 1.0
for _m in range(20):
    _P1[1, 20 + _m // 5, 4 + _m] = 1.0
    _P2[1, 4 + _m % 5, 4 + _m] = 1.0
_BONE = np.zeros((1, 32), np.float32)
_BONE[0, 24] = 1.0


QB = EB // 4


def _tp_body(*refs):
    eas = refs[0:4]
    xs = refs[4:8]
    shs = refs[8:12]
    w1_ref, b1_ref, w2_ref, b2_ref, r_ref, s_ref, p1_ref, p2_ref, bone_ref = refs[12:21]
    out_ref = refs[21]
    bone = bone_ref[...]
    ea = jnp.concatenate([r_[...] for r_ in eas], axis=0)
    x = jnp.concatenate([r_[...] for r_ in xs], axis=0)
    sh = jnp.concatenate([r_[...] for r_ in shs], axis=0)
    bf = jnp.bfloat16
    h = jnp.maximum(
        jnp.dot(ea.astype(bf), w1_ref[...].astype(bf),
                preferred_element_type=jnp.float32) + b1_ref[...], 0.0)
    w = jnp.dot(h.astype(bf), w2_ref[...].astype(bf),
                preferred_element_type=jnp.float32) + b2_ref[...]
    x2 = jnp.dot(x, r_ref[...], preferred_element_type=jnp.float32)
    c = jnp.dot(w * x2, s_ref[...], preferred_element_type=jnp.float32)
    lo = (jnp.dot(c, p1_ref[0], preferred_element_type=jnp.float32)
          * jnp.dot(sh, p2_ref[0], preferred_element_type=jnp.float32) + bone)
    hi = (jnp.dot(c, p1_ref[1], preferred_element_type=jnp.float32)
          * jnp.dot(sh, p2_ref[1], preferred_element_type=jnp.float32) + bone)
    out_ref[0] = jnp.concatenate(
        [lo[r * QB:(r + 1) * QB] for r in range(4)], axis=1)
    out_ref[1] = jnp.concatenate(
        [hi[r * QB:(r + 1) * QB] for r in range(4)], axis=1)


def _tc_tp(edge_attr, x, edge_sh, w1, b1, w2p, b2p):
    nblk = N_EDGES // EB
    specs = []
    for width in (EDGE_FEAT, NSF, 9):
        for r in range(4):
            specs.append(pl.BlockSpec((QB, width),
                                      lambda i, r=r: (4 * i + r, 0)))
    specs += [
        pl.BlockSpec((EDGE_FEAT, HID), lambda i: (0, 0)),
        pl.BlockSpec((1, HID), lambda i: (0, 0)),
        pl.BlockSpec((HID, WNUM), lambda i: (0, 0)),
        pl.BlockSpec((1, WNUM), lambda i: (0, 0)),
        pl.BlockSpec((NSF, WNUM), lambda i: (0, 0)),
        pl.BlockSpec((WNUM, 24), lambda i: (0, 0)),
        pl.BlockSpec((2, 24, 32), lambda i: (0, 0, 0)),
        pl.BlockSpec((2, 9, 32), lambda i: (0, 0, 0)),
        pl.BlockSpec((1, 32), lambda i: (0, 0)),
    ]
    return pl.pallas_call(
        _tp_body,
        grid=(nblk,),
        in_specs=specs,
        out_specs=pl.BlockSpec((2, QB, 128), lambda i: (0, i, 0)),
        out_shape=jax.ShapeDtypeStruct((2, N_EDGES // 4, 128), jnp.float32),
    )(edge_attr, edge_attr, edge_attr, edge_attr, x, x, x, x,
      edge_sh, edge_sh, edge_sh, edge_sh,
      w1, b1, w2p, b2p,
      jnp.asarray(_R), jnp.asarray(_S), jnp.asarray(_P1), jnp.asarray(_P2),
      jnp.asarray(_BONE))


def _scatter_body(tp_hbm, src_hbm, z_hbm, out_hbm, idx_v, tp_v, acc_s):
    c = lax.axis_index("c")
    s = lax.axis_index("s")
    pltpu.sync_copy(z_hbm, acc_s.at[pl.ds(s * ROWS_PER_TILE, ROWS_PER_TILE)])
    plsc.subcore_barrier()

    per_t = N_EDGES // 16
    base = s * per_t

    def step(i, _):
        off = base + i * CHC
        pltpu.sync_copy(src_hbm.at[pl.ds(off, CHC)], idx_v)
        pltpu.sync_copy(tp_hbm.at[c, pl.ds(off, CHC)], tp_v)
        pltpu.sync_copy(tp_v, acc_s.at[idx_v], add=True)
        return ()

    lax.fori_loop(0, per_t // CHC, step, ())
    plsc.subcore_barrier()
    pltpu.sync_copy(acc_s.at[pl.ds(s * ROWS_PER_TILE, ROWS_PER_TILE)],
                    out_hbm.at[c, pl.ds(s * ROWS_PER_TILE, ROWS_PER_TILE)])


def _sc_scatter(tp, edge_src, zrows):
    mesh = plsc.VectorSubcoreMesh(core_axis_name="c", subcore_axis_name="s")
    k = functools.partial(
        pl.kernel,
        out_type=jax.ShapeDtypeStruct((2, N_NODES, 32), jnp.float32),
        mesh=mesh,
        scratch_types=[
            pltpu.VMEM((CHC,), jnp.int32),
            pltpu.VMEM((CHC, 32), jnp.float32),
            pltpu.VMEM_SHARED((N_NODES, 32), jnp.float32),
        ],
        compiler_params=pltpu.CompilerParams(use_tc_tiling_on_sc=False),
    )(_scatter_body)
    return k(tp, edge_src, zrows)


def _mean_body(acc_ref, na_ref, out_ref):
    lo = acc_ref[0]
    hi = acc_ref[1]
    cnt = jnp.maximum(lo[:, 24:25], 1.0)
    sums = jnp.concatenate([lo[:, 0:24], hi[:, 0:24]], axis=1)
    res = jnp.concatenate(
        [na_ref[...], jnp.zeros((NB, 32), jnp.float32)], axis=1)
    out_ref[...] = sums / cnt + res


def _tc_mean(acc, node_attr):
    grid = (N_NODES // NB,)
    return pl.pallas_call(
        _mean_body,
        grid=grid,
        in_specs=[
            pl.BlockSpec((2, NB, 32), lambda i: (0, i, 0)),
            pl.BlockSpec((NB, NSF), lambda i: (i, 0)),
        ],
        out_specs=pl.BlockSpec((NB, 48), lambda i: (i, 0)),
        out_shape=jax.ShapeDtypeStruct((N_NODES, 48), jnp.float32),
    )(acc, node_attr)


def kernel(node_attr, edge_index, edge_attr, edge_sh, fc1_w, fc1_b, fc2_w, fc2_b):
    ei = edge_index.astype(jnp.int32)
    edge_src = ei[0]
    edge_dst = ei[1]
    w2p = jnp.take(fc2_w, _PERM, axis=1)
    b2p = jnp.take(fc2_b, _PERM).reshape(1, WNUM)
    b1 = fc1_b.reshape(1, HID)
    zrows = jnp.zeros((ROWS_PER_TILE, 32), jnp.float32)

    x = _sc_gather(node_attr, edge_dst)
    tp128 = _tc_tp(edge_attr, x, edge_sh, fc1_w, b1, w2p, b2p)
    tp = jnp.reshape(tp128, (2, N_EDGES, 32))
    src_perm = jnp.reshape(edge_src, (N_EDGES // EB, 4, QB)
                           ).transpose(0, 2, 1).reshape(N_EDGES)
    acc = _sc_scatter(tp, src_perm, zrows)
    return _tc_mean(acc, node_attr)


# speedup vs baseline: 1.2453x; 1.2453x over previous
import functools

import numpy as np
import jax
import jax.numpy as jnp
from jax import lax
from jax.experimental import pallas as pl
from jax.experimental.pallas import tpu as pltpu
from jax.experimental.pallas import tpu_sc as plsc

NSF = 16
EDGE_FEAT = 48
HID = 48
WNUM = 384
N_NODES = 50000
N_EDGES = 800000
EB = 3200
NB = 2000
CHA = 1000
CHC = 400
ROWS_PER_TILE = N_NODES // 16

_PERM = np.empty(WNUM, np.int32)
for _u in range(16):
    for _t in range(24):
        if _t < 16:
            _src = _u * 16 + _t
        elif _t < 20:
            _src = 256 + _u * 4 + (_t - 16)
        else:
            _src = 320 + _u * 4 + (_t - 20)
        _PERM[_u * 24 + _t] = _src


def _gather_body(node_hbm, dst_hbm, x_hbm, idx_v, rows_v, sem):
    c = lax.axis_index("c")
    s = lax.axis_index("s")
    wid = s * 2 + c
    per_w = N_EDGES // 32
    base = wid * per_w

    def step(i, _):
        off = base + i * CHA
        pltpu.sync_copy(dst_hbm.at[pl.ds(off, CHA)], idx_v)
        pltpu.async_copy(node_hbm.at[idx_v], rows_v, sem).wait()
        pltpu.sync_copy(rows_v, x_hbm.at[pl.ds(off, CHA)])
        return ()

    lax.fori_loop(0, per_w // CHA, step, ())


def _sc_gather(node_attr, edge_dst):
    mesh = plsc.VectorSubcoreMesh(core_axis_name="c", subcore_axis_name="s")
    k = functools.partial(
        pl.kernel,
        out_type=jax.ShapeDtypeStruct((N_EDGES, NSF), jnp.float32),
        mesh=mesh,
        scratch_types=[
            pltpu.VMEM((CHA,), jnp.int32),
            pltpu.VMEM((CHA, NSF), jnp.float32),
            pltpu.SemaphoreType.DMA,
        ],
        compiler_params=pltpu.CompilerParams(use_tc_tiling_on_sc=False),
    )(_gather_body)
    return k(node_attr, edge_dst)


_R = np.zeros((16, 384), np.float32)
_S = np.zeros((384, 24), np.float32)
for _u in range(16):
    for _t in range(24):
        _R[_u, _u * 24 + _t] = 1.0
        _S[_u * 24 + _t, _t] = 0.25
_P1 = np.zeros((2, 24, 32), np.float32)
_P2 = np.zeros((2, 9, 32), np.float32)
for _m in range(16):
    _P1[0, _m, _m] = 1.0
    _P2[0, 0, _m] = 1.0
for _m in range(12):
    _half, _col = (0, 16 + _m) if _m < 8 else (1, _m - 8)
    _P1[_half, 16 + _m // 3, _col] = 1.0
    _P2[_half, 1 + _m % 3, _col] = 1.0
for _m in range(20):
    _P1[1, 20 + _m // 5, 4 + _m] = 1.0
    _P2[1, 4 + _m % 5, 4 + _m] = 1.0
_BONE = np.zeros((1, 32), np.float32)
_BONE[0, 24] = 1.0


QB = EB // 4


def _tp_body(eat_hbm, sht_hbm, x0_ref, x1_ref, x2_ref, x3_ref,
             w1_ref, b1_ref, w2_ref, b2_ref,
             r_ref, s_ref, p1_ref, p2_ref, bone_ref,
             out_ref, ea_v, sh_v, sem):
    i = pl.program_id(0)
    n = pl.num_programs(0)

    def start_copies(blk, slot):
        pltpu.make_async_copy(eat_hbm.at[:, pl.ds(blk * EB, EB)],
                              ea_v.at[slot], sem.at[slot, 0]).start()
        pltpu.make_async_copy(sht_hbm.at[:, pl.ds(blk * EB, EB)],
                              sh_v.at[slot], sem.at[slot, 1]).start()

    slot = lax.rem(i, 2)

    @pl.when(i == 0)
    def _():
        start_copies(0, 0)

    @pl.when(i + 1 < n)
    def _():
        start_copies(i + 1, lax.rem(i + 1, 2))

    pltpu.make_async_copy(eat_hbm.at[:, pl.ds(i * EB, EB)],
                          ea_v.at[slot], sem.at[slot, 0]).wait()
    pltpu.make_async_copy(sht_hbm.at[:, pl.ds(i * EB, EB)],
                          sh_v.at[slot], sem.at[slot, 1]).wait()

    eat = ea_v[slot]
    x = jnp.concatenate([x0_ref[...], x1_ref[...], x2_ref[...], x3_ref[...]],
                        axis=0)
    sht = sh_v[slot]
    bone = bone_ref[...]
    tdn = (((0,), (0,)), ((), ()))
    h = jnp.maximum(
        lax.dot_general(eat, w1_ref[...], tdn,
                        preferred_element_type=jnp.float32) + b1_ref[...], 0.0)
    w = jnp.dot(h, w2_ref[...], preferred_element_type=jnp.float32) + b2_ref[...]
    x2 = jnp.dot(x, r_ref[...], preferred_element_type=jnp.float32)
    c = jnp.dot(w * x2, s_ref[...], preferred_element_type=jnp.float32)
    lo = (jnp.dot(c, p1_ref[0], preferred_element_type=jnp.float32)
          * lax.dot_general(sht, p2_ref[0], tdn,
                            preferred_element_type=jnp.float32) + bone)
    hi = (jnp.dot(c, p1_ref[1], preferred_element_type=jnp.float32)
          * lax.dot_general(sht, p2_ref[1], tdn,
                            preferred_element_type=jnp.float32) + bone)
    out_ref[0] = jnp.concatenate(
        [lo[r * QB:(r + 1) * QB] for r in range(4)], axis=1)
    out_ref[1] = jnp.concatenate(
        [hi[r * QB:(r + 1) * QB] for r in range(4)], axis=1)


def _tc_tp(ea_t, x, sh_t, w1, b1, w2p, b2p):
    nblk = N_EDGES // EB
    specs = [
        pl.BlockSpec(memory_space=pl.ANY),
        pl.BlockSpec(memory_space=pl.ANY),
    ] + [
        pl.BlockSpec((QB, NSF), lambda i, r=r: (4 * i + r, 0)) for r in range(4)
    ] + [
        pl.BlockSpec((EDGE_FEAT, HID), lambda i: (0, 0)),
        pl.BlockSpec((1, HID), lambda i: (0, 0)),
        pl.BlockSpec((HID, WNUM), lambda i: (0, 0)),
        pl.BlockSpec((1, WNUM), lambda i: (0, 0)),
        pl.BlockSpec((NSF, WNUM), lambda i: (0, 0)),
        pl.BlockSpec((WNUM, 24), lambda i: (0, 0)),
        pl.BlockSpec((2, 24, 32), lambda i: (0, 0, 0)),
        pl.BlockSpec((2, 9, 32), lambda i: (0, 0, 0)),
        pl.BlockSpec((1, 32), lambda i: (0, 0)),
    ]
    return pl.pallas_call(
        _tp_body,
        grid=(nblk,),
        in_specs=specs,
        out_specs=pl.BlockSpec((2, QB, 128), lambda i: (0, i, 0)),
        out_shape=jax.ShapeDtypeStruct((2, N_EDGES // 4, 128), jnp.float32),
        scratch_shapes=[
            pltpu.VMEM((2, EDGE_FEAT, EB), jnp.float32),
            pltpu.VMEM((2, 9, EB), jnp.float32),
            pltpu.SemaphoreType.DMA((2, 2)),
        ],
    )(ea_t, sh_t, x, x, x, x,
      w1, b1, w2p, b2p,
      jnp.asarray(_R), jnp.asarray(_S), jnp.asarray(_P1), jnp.asarray(_P2),
      jnp.asarray(_BONE))


def _scatter_body(tp_hbm, src_hbm, z_hbm, out_hbm, idx_v, tp_v, acc_s):
    c = lax.axis_index("c")
    s = lax.axis_index("s")
    pltpu.sync_copy(z_hbm, acc_s.at[pl.ds(s * ROWS_PER_TILE, ROWS_PER_TILE)])
    plsc.subcore_barrier()

    per_t = N_EDGES // 16
    base = s * per_t

    def step(i, _):
        off = base + i * CHC
        pltpu.sync_copy(src_hbm.at[pl.ds(off, CHC)], idx_v)
        pltpu.sync_copy(tp_hbm.at[c, pl.ds(off, CHC)], tp_v)
        pltpu.sync_copy(tp_v, acc_s.at[idx_v], add=True)
        return ()

    lax.fori_loop(0, per_t // CHC, step, ())
    plsc.subcore_barrier()
    pltpu.sync_copy(acc_s.at[pl.ds(s * ROWS_PER_TILE, ROWS_PER_TILE)],
                    out_hbm.at[c, pl.ds(s * ROWS_PER_TILE, ROWS_PER_TILE)])


def _sc_scatter(tp, edge_src, zrows):
    mesh = plsc.VectorSubcoreMesh(core_axis_name="c", subcore_axis_name="s")
    k = functools.partial(
        pl.kernel,
        out_type=jax.ShapeDtypeStruct((2, N_NODES, 32), jnp.float32),
        mesh=mesh,
        scratch_types=[
            pltpu.VMEM((CHC,), jnp.int32),
            pltpu.VMEM((CHC, 32), jnp.float32),
            pltpu.VMEM_SHARED((N_NODES, 32), jnp.float32),
        ],
        compiler_params=pltpu.CompilerParams(use_tc_tiling_on_sc=False),
    )(_scatter_body)
    return k(tp, edge_src, zrows)


def _mean_body(acc_ref, na_ref, out_ref):
    lo = acc_ref[0]
    hi = acc_ref[1]
    cnt = jnp.maximum(lo[:, 24:25], 1.0)
    sums = jnp.concatenate([lo[:, 0:24], hi[:, 0:24]], axis=1)
    res = jnp.concatenate(
        [na_ref[...], jnp.zeros((NB, 32), jnp.float32)], axis=1)
    out_ref[...] = sums / cnt + res


def _tc_mean(acc, node_attr):
    grid = (N_NODES // NB,)
    return pl.pallas_call(
        _mean_body,
        grid=grid,
        in_specs=[
            pl.BlockSpec((2, NB, 32), lambda i: (0, i, 0)),
            pl.BlockSpec((NB, NSF), lambda i: (i, 0)),
        ],
        out_specs=pl.BlockSpec((NB, 48), lambda i: (i, 0)),
        out_shape=jax.ShapeDtypeStruct((N_NODES, 48), jnp.float32),
    )(acc, node_attr)


def kernel(node_attr, edge_index, edge_attr, edge_sh, fc1_w, fc1_b, fc2_w, fc2_b):
    ei = edge_index.astype(jnp.int32)
    edge_src = ei[0]
    edge_dst = ei[1]
    w2p = jnp.take(fc2_w, _PERM, axis=1)
    b2p = jnp.take(fc2_b, _PERM).reshape(1, WNUM)
    b1 = fc1_b.reshape(1, HID)
    zrows = jnp.zeros((ROWS_PER_TILE, 32), jnp.float32)

    x = _sc_gather(node_attr, edge_dst)
    tp128 = _tc_tp(edge_attr.T, x, edge_sh.T, fc1_w, b1, w2p, b2p)
    tp = jnp.reshape(tp128, (2, N_EDGES, 32))
    src_perm = jnp.reshape(edge_src, (N_EDGES // EB, 4, QB)
                           ).transpose(0, 2, 1).reshape(N_EDGES)
    acc = _sc_scatter(tp, src_perm, zrows)
    return _tc_mean(acc, node_attr)


# speedup vs baseline: 1.3699x; 1.1001x over previous
import functools

import numpy as np
import jax
import jax.numpy as jnp
from jax import lax
from jax.experimental import pallas as pl
from jax.experimental.pallas import tpu as pltpu
from jax.experimental.pallas import tpu_sc as plsc

NSF = 16
EDGE_FEAT = 48
HID = 48
WNUM = 384
N_NODES = 50000
N_EDGES = 800000
EB = 3200
NB = 2000
CHA = 1000
CHC = 400
ROWS_PER_TILE = N_NODES // 16

_PERM = np.empty(WNUM, np.int32)
for _u in range(16):
    for _t in range(24):
        if _t < 16:
            _src = _u * 16 + _t
        elif _t < 20:
            _src = 256 + _u * 4 + (_t - 16)
        else:
            _src = 320 + _u * 4 + (_t - 20)
        _PERM[_u * 24 + _t] = _src


def _gather_body(node_hbm, dst_hbm, x_hbm, idx_v, rows_v, sem):
    c = lax.axis_index("c")
    s = lax.axis_index("s")
    wid = s * 2 + c
    per_w = N_EDGES // 32
    base = wid * per_w

    def step(i, _):
        off = base + i * CHA
        pltpu.sync_copy(dst_hbm.at[pl.ds(off, CHA)], idx_v)
        pltpu.async_copy(node_hbm.at[idx_v], rows_v, sem).wait()
        pltpu.sync_copy(rows_v, x_hbm.at[pl.ds(off, CHA)])
        return ()

    lax.fori_loop(0, per_w // CHA, step, ())


def _sc_gather(node_attr, edge_dst):
    mesh = plsc.VectorSubcoreMesh(core_axis_name="c", subcore_axis_name="s")
    k = functools.partial(
        pl.kernel,
        out_type=jax.ShapeDtypeStruct((N_EDGES, NSF), jnp.float32),
        mesh=mesh,
        scratch_types=[
            pltpu.VMEM((CHA,), jnp.int32),
            pltpu.VMEM((CHA, NSF), jnp.float32),
            pltpu.SemaphoreType.DMA,
        ],
        compiler_params=pltpu.CompilerParams(use_tc_tiling_on_sc=False),
    )(_gather_body)
    return k(node_attr, edge_dst)


_R = np.zeros((16, 384), np.float32)
_S = np.zeros((384, 24), np.float32)
for _u in range(16):
    for _t in range(24):
        _R[_u, _u * 24 + _t] = 1.0
        _S[_u * 24 + _t, _t] = 0.25
_P1 = np.zeros((2, 24, 32), np.float32)
_P2 = np.zeros((2, 9, 32), np.float32)
for _m in range(16):
    _P1[0, _m, _m] = 1.0
    _P2[0, 0, _m] = 1.0
for _m in range(12):
    _half, _col = (0, 16 + _m) if _m < 8 else (1, _m - 8)
    _P1[_half, 16 + _m // 3, _col] = 1.0
    _P2[_half, 1 + _m % 3, _col] = 1.0
for _m in range(20):
    _P1[1, 20 + _m // 5, 4 + _m] = 1.0
    _P2[1, 4 + _m % 5, 4 + _m] = 1.0
_BONE = np.zeros((1, 32), np.float32)
_BONE[0, 24] = 1.0


QB = EB // 4


def _tp_body(eat_hbm, sht_hbm, x0_ref, x1_ref, x2_ref, x3_ref,
             w1_ref, b1_ref, w2_ref, b2_ref,
             r_ref, s_ref, p1_ref, p2_ref, bone_ref,
             out_ref, ea_v, sh_v, sem):
    i = pl.program_id(0)
    n = pl.num_programs(0)

    def start_copies(blk, slot):
        pltpu.make_async_copy(eat_hbm.at[:, pl.ds(blk * EB, EB)],
                              ea_v.at[slot], sem.at[slot, 0]).start()
        pltpu.make_async_copy(sht_hbm.at[:, pl.ds(blk * EB, EB)],
                              sh_v.at[slot], sem.at[slot, 1]).start()

    slot = lax.rem(i, 2)

    @pl.when(i == 0)
    def _():
        start_copies(0, 0)

    @pl.when(i + 1 < n)
    def _():
        start_copies(i + 1, lax.rem(i + 1, 2))

    pltpu.make_async_copy(eat_hbm.at[:, pl.ds(i * EB, EB)],
                          ea_v.at[slot], sem.at[slot, 0]).wait()
    pltpu.make_async_copy(sht_hbm.at[:, pl.ds(i * EB, EB)],
                          sh_v.at[slot], sem.at[slot, 1]).wait()

    eat = ea_v[slot]
    x = jnp.concatenate([x0_ref[...], x1_ref[...], x2_ref[...], x3_ref[...]],
                        axis=0)
    sht = sh_v[slot]
    bone = bone_ref[...]
    tdn = (((0,), (0,)), ((), ()))
    h = jnp.maximum(
        lax.dot_general(eat, w1_ref[...], tdn,
                        preferred_element_type=jnp.float32) + b1_ref[...], 0.0)
    w = jnp.dot(h, w2_ref[...], preferred_element_type=jnp.float32) + b2_ref[...]
    x2 = jnp.dot(x, r_ref[...], preferred_element_type=jnp.float32)
    c = jnp.dot(w * x2, s_ref[...], preferred_element_type=jnp.float32)
    lo = (jnp.dot(c, p1_ref[0], preferred_element_type=jnp.float32)
          * lax.dot_general(sht, p2_ref[0], tdn,
                            preferred_element_type=jnp.float32) + bone)
    hi = (jnp.dot(c, p1_ref[1], preferred_element_type=jnp.float32)
          * lax.dot_general(sht, p2_ref[1], tdn,
                            preferred_element_type=jnp.float32) + bone)
    out_ref[0] = jnp.concatenate(
        [lo[r * QB:(r + 1) * QB] for r in range(4)], axis=1)
    out_ref[1] = jnp.concatenate(
        [hi[r * QB:(r + 1) * QB] for r in range(4)], axis=1)


def _tc_tp(ea_t, x, sh_t, w1, b1, w2p, b2p):
    nblk = N_EDGES // EB
    specs = [
        pl.BlockSpec(memory_space=pl.ANY),
        pl.BlockSpec(memory_space=pl.ANY),
    ] + [
        pl.BlockSpec((QB, NSF), lambda i, r=r: (4 * i + r, 0)) for r in range(4)
    ] + [
        pl.BlockSpec((EDGE_FEAT, HID), lambda i: (0, 0)),
        pl.BlockSpec((1, HID), lambda i: (0, 0)),
        pl.BlockSpec((HID, WNUM), lambda i: (0, 0)),
        pl.BlockSpec((1, WNUM), lambda i: (0, 0)),
        pl.BlockSpec((NSF, WNUM), lambda i: (0, 0)),
        pl.BlockSpec((WNUM, 24), lambda i: (0, 0)),
        pl.BlockSpec((2, 24, 32), lambda i: (0, 0, 0)),
        pl.BlockSpec((2, 9, 32), lambda i: (0, 0, 0)),
        pl.BlockSpec((1, 32), lambda i: (0, 0)),
    ]
    return pl.pallas_call(
        _tp_body,
        grid=(nblk,),
        in_specs=specs,
        out_specs=pl.BlockSpec((2, QB, 128), lambda i: (0, i, 0)),
        out_shape=jax.ShapeDtypeStruct((2, N_EDGES // 4, 128), jnp.float32),
        scratch_shapes=[
            pltpu.VMEM((2, EDGE_FEAT, EB), jnp.float32),
            pltpu.VMEM((2, 9, EB), jnp.float32),
            pltpu.SemaphoreType.DMA((2, 2)),
        ],
    )(ea_t, sh_t, x, x, x, x,
      w1, b1, w2p, b2p,
      jnp.asarray(_R), jnp.asarray(_S), jnp.asarray(_P1), jnp.asarray(_P2),
      jnp.asarray(_BONE))


def _scatter_body(tp_hbm, src_hbm, z_hbm, out_hbm, idx_v, tp_v, acc_s, sem):
    c = lax.axis_index("c")
    s = lax.axis_index("s")
    pltpu.sync_copy(z_hbm, acc_s.at[pl.ds(s * ROWS_PER_TILE, ROWS_PER_TILE)])
    plsc.subcore_barrier()

    per_t = N_EDGES // 16
    base = s * per_t
    n = per_t // CHC

    def start(i, slot):
        off = base + i * CHC
        pltpu.async_copy(src_hbm.at[pl.ds(off, CHC)], idx_v.at[slot],
                         sem.at[slot, 0])
        pltpu.async_copy(tp_hbm.at[c, pl.ds(off, CHC)], tp_v.at[slot],
                         sem.at[slot, 1])

    def wait(i, slot):
        off = base + i * CHC
        pltpu.make_async_copy(src_hbm.at[pl.ds(off, CHC)], idx_v.at[slot],
                              sem.at[slot, 0]).wait()
        pltpu.make_async_copy(tp_hbm.at[c, pl.ds(off, CHC)], tp_v.at[slot],
                              sem.at[slot, 1]).wait()

    start(0, 0)

    def step(i, _):
        slot = lax.rem(i, 2)

        @pl.when(i + 1 < n)
        def _():
            start(i + 1, lax.rem(i + 1, 2))

        wait(i, slot)
        pltpu.sync_copy(tp_v.at[slot], acc_s.at[idx_v.at[slot]], add=True)
        return ()

    lax.fori_loop(0, n, step, ())
    plsc.subcore_barrier()
    pltpu.sync_copy(acc_s.at[pl.ds(s * ROWS_PER_TILE, ROWS_PER_TILE)],
                    out_hbm.at[c, pl.ds(s * ROWS_PER_TILE, ROWS_PER_TILE)])


def _sc_scatter(tp, edge_src, zrows):
    mesh = plsc.VectorSubcoreMesh(core_axis_name="c", subcore_axis_name="s")
    k = functools.partial(
        pl.kernel,
        out_type=jax.ShapeDtypeStruct((2, N_NODES, 32), jnp.float32),
        mesh=mesh,
        scratch_types=[
            pltpu.VMEM((2, CHC), jnp.int32),
            pltpu.VMEM((2, CHC, 32), jnp.float32),
            pltpu.VMEM_SHARED((N_NODES, 32), jnp.float32),
            pltpu.SemaphoreType.DMA((2, 2)),
        ],
        compiler_params=pltpu.CompilerParams(use_tc_tiling_on_sc=False),
    )(_scatter_body)
    return k(tp, edge_src, zrows)


def _mean_body(acc_ref, na_ref, out_ref):
    lo = acc_ref[0]
    hi = acc_ref[1]
    cnt = jnp.maximum(lo[:, 24:25], 1.0)
    sums = jnp.concatenate([lo[:, 0:24], hi[:, 0:24]], axis=1)
    res = jnp.concatenate(
        [na_ref[...], jnp.zeros((NB, 32), jnp.float32)], axis=1)
    out_ref[...] = sums / cnt + res


def _tc_mean(acc, node_attr):
    grid = (N_NODES // NB,)
    return pl.pallas_call(
        _mean_body,
        grid=grid,
        in_specs=[
            pl.BlockSpec((2, NB, 32), lambda i: (0, i, 0)),
            pl.BlockSpec((NB, NSF), lambda i: (i, 0)),
        ],
        out_specs=pl.BlockSpec((NB, 48), lambda i: (i, 0)),
        out_shape=jax.ShapeDtypeStruct((N_NODES, 48), jnp.float32),
    )(acc, node_attr)


def kernel(node_attr, edge_index, edge_attr, edge_sh, fc1_w, fc1_b, fc2_w, fc2_b):
    ei = edge_index.astype(jnp.int32)
    edge_src = ei[0]
    edge_dst = ei[1]
    w2p = jnp.take(fc2_w, _PERM, axis=1)
    b2p = jnp.take(fc2_b, _PERM).reshape(1, WNUM)
    b1 = fc1_b.reshape(1, HID)
    zrows = jnp.zeros((ROWS_PER_TILE, 32), jnp.float32)

    x = _sc_gather(node_attr, edge_dst)
    tp128 = _tc_tp(edge_attr.T, x, edge_sh.T, fc1_w, b1, w2p, b2p)
    tp = jnp.reshape(tp128, (2, N_EDGES, 32))
    src_perm = jnp.reshape(edge_src, (N_EDGES // EB, 4, QB)
                           ).transpose(0, 2, 1).reshape(N_EDGES)
    acc = _sc_scatter(tp, src_perm, zrows)
    return _tc_mean(acc, node_attr)
